# baseline (speedup 1.0000x reference)
import numpy as np
import jax
import jax.numpy as jnp
from jax import lax
from jax.experimental import pallas as pl
from jax.experimental.pallas import tpu as pltpu, tpu_sc as plsc

NEURONS_C = 32768
K_C = 64
BATCH_C = 128
NWORKERS = 32
ROWS_PER_WORKER = BATCH_C // NWORKERS
NV_ROW = NEURONS_C // 16

MIN32 = np.int32(-2**31)
M7F = np.int32(0x7FFFFFFF)
NEG_INF = np.float32(-np.inf)


def _keyf(u):
    sk = u ^ MIN32
    return lax.bitcast_convert_type(
        sk ^ (lax.shift_right_arithmetic(sk, 31) & M7F), jnp.float32)


def _sc_body(s_hbm, o_hbm, row_v, hist_v, suf_v, cv_v, ci_v, wv_v, wi_v,
             zero_spm, sem):
    cid = lax.axis_index("c")
    sid = lax.axis_index("s")
    wid = sid * 2 + cid
    lane = lax.iota(jnp.int32, 16)
    lane_base = lane * 256
    ones16 = jnp.ones((16,), jnp.int32)
    zeros16 = jnp.zeros((16,), jnp.int32)
    fzeros16 = jnp.zeros((16,), jnp.float32)

    @pl.when(sid == 0)
    def _fill_zero():
        @plsc.parallel_loop(0, NV_ROW, unroll=8)
        def _z(i):
            row_v[pl.ds(i * 16, 16)] = fzeros16
        pltpu.sync_copy(row_v, zero_spm)

    plsc.subcore_barrier()

    def do_row(row):
        zdma = pltpu.make_async_copy(
            zero_spm, o_hbm.at[pl.ds(row * NEURONS_C, NEURONS_C)], sem)
        zdma.start()
        pltpu.sync_copy(s_hbm.at[row], row_v)

        @plsc.parallel_loop(0, 256, unroll=8)
        def _zh(i):
            hist_v[pl.ds(i * 16, 16)] = zeros16

        @plsc.parallel_loop(0, NV_ROW, unroll=8)
        def _h(i):
            x = row_v[pl.ds(i * 16, 16)]
            iv = lax.bitcast_convert_type(x, jnp.int32)
            uk = iv ^ (lax.shift_right_arithmetic(iv, 31) | MIN32)
            bin_ = lax.shift_right_logical(uk, 24)
            plsc.addupdate_scatter(hist_v, [lane_base + bin_], ones16)

        suf_v[pl.ds(256, 16)] = zeros16
        running = jnp.int32(0)
        for c in range(15, -1, -1):
            tot = hist_v[pl.ds(c * 16, 16)]
            for j in range(1, 16):
                tot = tot + hist_v[pl.ds(j * 256 + c * 16, 16)]
            within = lax.rev(plsc.cumsum(lax.rev(tot, (0,))), (0,))
            suf_v[pl.ds(c * 16, 16)] = within + running
            running = running + jnp.sum(tot)

        b0 = zeros16
        for b in (128, 64, 32, 16, 8, 4, 2, 1):
            cand = b0 | np.int32(b)
            b0 = jnp.where(plsc.load_gather(suf_v, [cand]) >= K_C, cand, b0)
        c_above = plsc.load_gather(suf_v, [b0 + 1])
        k_rem = K_C - c_above

        @plsc.parallel_loop(0, NV_ROW, unroll=4, carry=zeros16)
        def off(i, off_c):
            x = row_v[pl.ds(i * 16, 16)]
            iv = lax.bitcast_convert_type(x, jnp.int32)
            uk = iv ^ (lax.shift_right_arithmetic(iv, 31) | MIN32)
            m = lax.shift_right_logical(uk, 24) == b0
            pos = off_c + plsc.cumsum(m.astype(jnp.int32)) - 1
            plsc.store_scatter(cv_v, [pos], x, mask=m)
            plsc.store_scatter(ci_v, [pos], lane + i * 16, mask=m)
            return off_c + plsc.all_reduce_population_count(m)

        for t in range(4):
            plsc.store_scatter(cv_v, [off + lane + t * 16],
                               jnp.full((16,), NEG_INF))
        nv4 = (((off + 63) >> 6)[0]).astype(jnp.int32)

        def count_ge(t_f):
            @plsc.parallel_loop(0, nv4, carry=zeros16)
            def acc(t, acc_c):
                for q in range(4):
                    m = cv_v[pl.ds(t * 64 + q * 16, 16)] >= t_f
                    acc_c = acc_c + plsc.all_reduce_population_count(m)
                return acc_c
            return acc

        u = lax.shift_left(b0, 24)
        for b in range(23, -1, -1):
            cand_u = u | np.int32(1 << b)
            cnt = count_ge(_keyf(cand_u))
            u = jnp.where(cnt >= k_rem, cand_u, u)
        thr_f = _keyf(u)

        def count_gt(t_f):
            @plsc.parallel_loop(0, nv4, carry=zeros16)
            def acc(t, acc_c):
                for q in range(4):
                    m = cv_v[pl.ds(t * 64 + q * 16, 16)] > t_f
                    acc_c = acc_c + plsc.all_reduce_population_count(m)
                return acc_c
            return acc

        n_gt = count_gt(thr_f)
        need = k_rem - n_gt
        n_eq = count_ge(thr_f) - n_gt

        def tie_search():
            jcut0 = zeros16
            for b in range(14, -1, -1):
                candj = jcut0 | np.int32(1 << b)

                @plsc.parallel_loop(0, nv4, carry=zeros16)
                def cnt_j(t, c_c):
                    for q in range(4):
                        m = (cv_v[pl.ds(t * 64 + q * 16, 16)] == thr_f) & (
                            ci_v[pl.ds(t * 64 + q * 16, 16)] >= candj)
                        c_c = c_c + plsc.all_reduce_population_count(m)
                    return c_c
                jcut0 = jnp.where(cnt_j >= need, candj, jcut0)
            return jcut0

        jcut = lax.cond(((n_eq != need).astype(jnp.int32))[0] != 0,
                        tie_search, lambda: zeros16)

        row_base = row * NEURONS_C

        @plsc.parallel_loop(0, nv4, carry=zeros16)
        def wcnt(t, w_c):
            for q in range(4):
                x = cv_v[pl.ds(t * 64 + q * 16, 16)]
                ix = ci_v[pl.ds(t * 64 + q * 16, 16)]
                m = (x > thr_f) | ((x == thr_f) & (ix >= jcut))
                pos = w_c + plsc.cumsum(m.astype(jnp.int32)) - 1
                plsc.store_scatter(wv_v, [pos], x, mask=m)
                plsc.store_scatter(wi_v, [pos], ix + row_base, mask=m)
                w_c = w_c + plsc.all_reduce_population_count(m)
            return w_c

        zdma.wait()
        wdma = pltpu.make_async_copy(wv_v, o_hbm.at[wi_v], sem)
        wdma.start()
        wdma.wait()

    for j in range(ROWS_PER_WORKER):
        do_row(wid * ROWS_PER_WORKER + j)


@jax.jit
def kernel(s):
    mesh = plsc.VectorSubcoreMesh(core_axis_name="c", subcore_axis_name="s",
                                  num_cores=2, num_subcores=16)
    out = pl.kernel(
        _sc_body,
        out_type=jax.ShapeDtypeStruct((BATCH_C * NEURONS_C,), jnp.float32),
        mesh=mesh,
        compiler_params=pltpu.CompilerParams(needs_layout_passes=False),
        scratch_types=[
            pltpu.VMEM((NEURONS_C,), jnp.float32),
            pltpu.VMEM((4096,), jnp.int32),
            pltpu.VMEM((272,), jnp.int32),
            pltpu.VMEM((NEURONS_C + 64,), jnp.float32),
            pltpu.VMEM((NEURONS_C + 64,), jnp.int32),
            pltpu.VMEM((K_C,), jnp.float32),
            pltpu.VMEM((K_C,), jnp.int32),
            pltpu.VMEM_SHARED((NEURONS_C,), jnp.float32),
            pltpu.SemaphoreType.DMA,
        ],
    )(s)
    return out.reshape(BATCH_C, NEURONS_C)


# speedup vs baseline: 1.9921x; 1.9921x over previous
import numpy as np
import jax
import jax.numpy as jnp
from jax import lax
from jax.experimental import pallas as pl
from jax.experimental.pallas import tpu as pltpu, tpu_sc as plsc

NEURONS_C = 32768
K_C = 64
BATCH_C = 128
NWORKERS = 32
ROWS_PER_WORKER = BATCH_C // NWORKERS
NV_ROW = NEURONS_C // 16
NSEG = NV_ROW // 16

MIN32 = np.int32(-2**31)
M7F = np.int32(0x7FFFFFFF)
NEG_INF = np.float32(-np.inf)


def _keyf(u):
    sk = u ^ MIN32
    return lax.bitcast_convert_type(
        sk ^ (lax.shift_right_arithmetic(sk, 31) & M7F), jnp.float32)


def _sc_body(s_hbm, o_hbm, row_v, gmax_v, cmax_v, glist_v, cv_v):
    wid = lax.axis_index("s") * 2 + lax.axis_index("c")
    lane = lax.iota(jnp.int32, 16)
    lane16 = lane * 16
    zeros16 = jnp.zeros((16,), jnp.int32)
    ninf16 = jnp.full((16,), NEG_INF)

    def do_row(row):
        pltpu.sync_copy(s_hbm.at[row], row_v)

        @plsc.parallel_loop(0, NSEG, unroll=2)
        def _g(t):
            acc = row_v[pl.ds(t * 256, 16)]
            for k in range(1, 16):
                acc = jnp.maximum(acc, row_v[pl.ds(t * 256 + k * 16, 16)])
            gmax_v[pl.ds(t * 16, 16)] = acc

        for t in range(16):
            acc = gmax_v[pl.ds(t * 128, 16)]
            for k in range(1, 8):
                acc = jnp.maximum(acc, gmax_v[pl.ds(t * 128 + k * 16, 16)])
            cmax_v[pl.ds(t * 16, 16)] = acc

        u = zeros16
        for b in range(31, -1, -1):
            bit = MIN32 if b == 31 else np.int32(1 << b)
            t_f = _keyf(u | bit)

            @plsc.parallel_loop(0, 4, carry=zeros16)
            def cnt(t, acc_c):
                for q in range(4):
                    m = cmax_v[pl.ds(t * 64 + q * 16, 16)] >= t_f
                    acc_c = acc_c + plsc.all_reduce_population_count(m)
                return acc_c
            u = jnp.where(cnt >= K_C, u | bit, u)
        c64_f = _keyf(u)

        @plsc.parallel_loop(0, NSEG, unroll=4, carry=zeros16)
        def ng(t, c_c):
            m = gmax_v[pl.ds(t * 16, 16)] >= c64_f
            pos = c_c + plsc.cumsum(m.astype(jnp.int32)) - 1
            plsc.store_scatter(glist_v, [pos], lane + t * 16, mask=m)
            return c_c + plsc.all_reduce_population_count(m)
        ng_s = ng[0]

        @plsc.parallel_loop(0, ng_s, unroll=2)
        def _cg(g):
            gid = plsc.load_gather(glist_v, [g + zeros16])
            base = lax.shift_left(lax.shift_right_arithmetic(gid, 4), 8)
            idx = base + lane16 + (gid & 15)
            cv_v[pl.ds(g * 16, 16)] = plsc.load_gather(row_v, [idx])

        for t in range(3):
            cv_v[pl.ds((ng_s + t) * 16, 16)] = ninf16
        nv4 = (ng_s + 3) >> 2

        def count_cmp(t_f, strict=False):
            @plsc.parallel_loop(0, nv4, carry=zeros16)
            def acc(t, acc_c):
                for q in range(4):
                    x = cv_v[pl.ds(t * 64 + q * 16, 16)]
                    m = (x > t_f) if strict else (x >= t_f)
                    acc_c = acc_c + plsc.all_reduce_population_count(m)
                return acc_c
            return acc

        u = zeros16
        for b in range(31, -1, -1):
            bit = MIN32 if b == 31 else np.int32(1 << b)
            cnt = count_cmp(_keyf(u | bit))
            u = jnp.where(cnt >= K_C, u | bit, u)
        thr_f = _keyf(u)

        n_gt = count_cmp(thr_f, strict=True)
        need = K_C - n_gt
        n_eq = count_cmp(thr_f) - n_gt

        def tie_search():
            jcut0 = zeros16
            for b in range(14, -1, -1):
                candj = jcut0 | np.int32(1 << b)

                @plsc.parallel_loop(0, NV_ROW, unroll=4, carry=zeros16)
                def cj(i, c_c):
                    x = row_v[pl.ds(i * 16, 16)]
                    m = (x == thr_f) & (lane + i * 16 >= candj)
                    return c_c + plsc.all_reduce_population_count(m)
                jcut0 = jnp.where(cj >= need, candj, jcut0)
            return jcut0

        tie_mode = ((n_eq != need).astype(jnp.int32))[0] != 0
        jcut = lax.cond(tie_mode, tie_search, lambda: zeros16)

        @pl.when(jnp.logical_not(tie_mode))
        def _mask_fast():
            @plsc.parallel_loop(0, NV_ROW, unroll=8)
            def _mk(i):
                x = row_v[pl.ds(i * 16, 16)]
                row_v[pl.ds(i * 16, 16)] = jnp.where(x >= thr_f, x, 0.0)

        @pl.when(tie_mode)
        def _mask_tie():
            @plsc.parallel_loop(0, NV_ROW, unroll=4)
            def _mk(i):
                x = row_v[pl.ds(i * 16, 16)]
                keep = (x > thr_f) | ((x == thr_f) & (lane + i * 16 >= jcut))
                row_v[pl.ds(i * 16, 16)] = jnp.where(keep, x, 0.0)

        pltpu.sync_copy(row_v, o_hbm.at[row])

    def _row_step(j, _):
        do_row(wid * ROWS_PER_WORKER + j)
        return 0
    lax.fori_loop(0, ROWS_PER_WORKER, _row_step, 0)


@jax.jit
def kernel(s):
    mesh = plsc.VectorSubcoreMesh(core_axis_name="c", subcore_axis_name="s",
                                  num_cores=2, num_subcores=16)
    return pl.kernel(
        _sc_body,
        out_type=jax.ShapeDtypeStruct((BATCH_C, NEURONS_C), jnp.float32),
        mesh=mesh,
        compiler_params=pltpu.CompilerParams(needs_layout_passes=False),
        scratch_types=[
            pltpu.VMEM((NEURONS_C,), jnp.float32),
            pltpu.VMEM((NV_ROW,), jnp.float32),
            pltpu.VMEM((256,), jnp.float32),
            pltpu.VMEM((NV_ROW + 16,), jnp.int32),
            pltpu.VMEM((NEURONS_C + 64,), jnp.float32),
        ],
    )(s)


# speedup vs baseline: 2.0828x; 1.0455x over previous
import numpy as np
import jax
import jax.numpy as jnp
from jax import lax
from jax.experimental import pallas as pl
from jax.experimental.pallas import tpu as pltpu, tpu_sc as plsc

NEURONS_C = 32768
K_C = 64
BATCH_C = 128
NWORKERS = 32
ROWS_PER_WORKER = BATCH_C // NWORKERS
NV_ROW = NEURONS_C // 16
NSEG = NV_ROW // 16

MIN32 = np.int32(-2**31)
M7F = np.int32(0x7FFFFFFF)
NEG_INF = np.float32(-np.inf)


def _keyf(u):
    sk = u ^ MIN32
    return lax.bitcast_convert_type(
        sk ^ (lax.shift_right_arithmetic(sk, 31) & M7F), jnp.float32)


def _sc_body(s_hbm, o_hbm, rowa_v, rowb_v, gmax_v, cmax_v, glist_v, cv_v,
             in0_sem, in1_sem, out0_sem, out1_sem):
    wid = lax.axis_index("s") * 2 + lax.axis_index("c")
    lane = lax.iota(jnp.int32, 16)
    lane16 = lane * 16
    zeros16 = jnp.zeros((16,), jnp.int32)
    ninf16 = jnp.full((16,), NEG_INF)

    def compute_row(buf, row):
        @plsc.parallel_loop(0, NSEG, unroll=2)
        def _g(t):
            acc = buf[pl.ds(t * 256, 16)]
            for k in range(1, 16):
                acc = jnp.maximum(acc, buf[pl.ds(t * 256 + k * 16, 16)])
            gmax_v[pl.ds(t * 16, 16)] = acc

        for t in range(16):
            acc = gmax_v[pl.ds(t * 128, 16)]
            for k in range(1, 8):
                acc = jnp.maximum(acc, gmax_v[pl.ds(t * 128 + k * 16, 16)])
            cmax_v[pl.ds(t * 16, 16)] = acc

        u = zeros16
        for b in range(31, -1, -1):
            bit = MIN32 if b == 31 else np.int32(1 << b)
            t_f = _keyf(u | bit)

            @plsc.parallel_loop(0, 4, carry=zeros16)
            def cnt(t, acc_c):
                for q in range(4):
                    m = cmax_v[pl.ds(t * 64 + q * 16, 16)] >= t_f
                    acc_c = acc_c + plsc.all_reduce_population_count(m)
                return acc_c
            u = jnp.where(cnt >= K_C, u | bit, u)
        c64_f = _keyf(u)

        @plsc.parallel_loop(0, NSEG, unroll=4, carry=zeros16)
        def ng(t, c_c):
            m = gmax_v[pl.ds(t * 16, 16)] >= c64_f
            pos = c_c + plsc.cumsum(m.astype(jnp.int32)) - 1
            plsc.store_scatter(glist_v, [pos], lane + t * 16, mask=m)
            return c_c + plsc.all_reduce_population_count(m)
        ng_s = ng[0]

        @plsc.parallel_loop(0, ng_s, unroll=2)
        def _cg(g):
            gid = plsc.load_gather(glist_v, [g + zeros16])
            base = lax.shift_left(lax.shift_right_arithmetic(gid, 4), 8)
            idx = base + lane16 + (gid & 15)
            cv_v[pl.ds(g * 16, 16)] = plsc.load_gather(buf, [idx])

        for t in range(3):
            cv_v[pl.ds((ng_s + t) * 16, 16)] = ninf16
        nv4 = (ng_s + 3) >> 2

        def count_cmp(t_f, strict=False):
            @plsc.parallel_loop(0, nv4, carry=zeros16)
            def acc(t, acc_c):
                for q in range(4):
                    x = cv_v[pl.ds(t * 64 + q * 16, 16)]
                    m = (x > t_f) if strict else (x >= t_f)
                    acc_c = acc_c + plsc.all_reduce_population_count(m)
                return acc_c
            return acc

        u = zeros16
        for b in range(31, -1, -1):
            bit = MIN32 if b == 31 else np.int32(1 << b)
            cnt = count_cmp(_keyf(u | bit))
            u = jnp.where(cnt >= K_C, u | bit, u)
        thr_f = _keyf(u)

        n_gt = count_cmp(thr_f, strict=True)
        need = K_C - n_gt
        n_eq = count_cmp(thr_f) - n_gt

        def tie_search():
            jcut0 = zeros16
            for b in range(14, -1, -1):
                candj = jcut0 | np.int32(1 << b)

                @plsc.parallel_loop(0, NV_ROW, unroll=4, carry=zeros16)
                def cj(i, c_c):
                    x = buf[pl.ds(i * 16, 16)]
                    m = (x == thr_f) & (lane + i * 16 >= candj)
                    return c_c + plsc.all_reduce_population_count(m)
                jcut0 = jnp.where(cj >= need, candj, jcut0)
            return jcut0

        tie_mode = ((n_eq != need).astype(jnp.int32))[0] != 0
        jcut = lax.cond(tie_mode, tie_search, lambda: zeros16)

        @pl.when(jnp.logical_not(tie_mode))
        def _mask_fast():
            @plsc.parallel_loop(0, NV_ROW, unroll=8)
            def _mk(i):
                x = buf[pl.ds(i * 16, 16)]
                buf[pl.ds(i * 16, 16)] = jnp.where(x >= thr_f, x, 0.0)

        @pl.when(tie_mode)
        def _mask_tie():
            @plsc.parallel_loop(0, NV_ROW, unroll=4)
            def _mk(i):
                x = buf[pl.ds(i * 16, 16)]
                keep = (x > thr_f) | ((x == thr_f) & (lane + i * 16 >= jcut))
                buf[pl.ds(i * 16, 16)] = jnp.where(keep, x, 0.0)

    base = wid * ROWS_PER_WORKER

    def in_dma(buf, row, sem):
        return pltpu.make_async_copy(s_hbm.at[row], buf, sem)

    def out_dma(buf, row, sem):
        return pltpu.make_async_copy(buf, o_hbm.at[row], sem)

    in_dma(rowa_v, base, in0_sem).start()

    def _pair(p, _):
        r0 = base + 2 * p
        in_dma(rowa_v, r0, in0_sem).wait()

        @pl.when(p > 0)
        def _w():
            out_dma(rowb_v, r0 - 1, out1_sem).wait()
        in_dma(rowb_v, r0 + 1, in1_sem).start()
        compute_row(rowa_v, r0)
        out_dma(rowa_v, r0, out0_sem).start()
        in_dma(rowb_v, r0 + 1, in1_sem).wait()
        compute_row(rowb_v, r0 + 1)
        out_dma(rowb_v, r0 + 1, out1_sem).start()
        out_dma(rowa_v, r0, out0_sem).wait()

        @pl.when(p == 0)
        def _n():
            in_dma(rowa_v, r0 + 2, in0_sem).start()
        return 0

    lax.fori_loop(0, ROWS_PER_WORKER // 2, _pair, 0)
    out_dma(rowb_v, base + 3, out1_sem).wait()


@jax.jit
def kernel(s):
    mesh = plsc.VectorSubcoreMesh(core_axis_name="c", subcore_axis_name="s",
                                  num_cores=2, num_subcores=16)
    return pl.kernel(
        _sc_body,
        out_type=jax.ShapeDtypeStruct((BATCH_C, NEURONS_C), jnp.float32),
        mesh=mesh,
        compiler_params=pltpu.CompilerParams(needs_layout_passes=False),
        scratch_types=[
            pltpu.VMEM((NEURONS_C,), jnp.float32),
            pltpu.VMEM((NEURONS_C,), jnp.float32),
            pltpu.VMEM((NV_ROW,), jnp.float32),
            pltpu.VMEM((256,), jnp.float32),
            pltpu.VMEM((NV_ROW + 16,), jnp.int32),
            pltpu.VMEM((NEURONS_C + 64,), jnp.float32),
            pltpu.SemaphoreType.DMA,
            pltpu.SemaphoreType.DMA,
            pltpu.SemaphoreType.DMA,
            pltpu.SemaphoreType.DMA,
        ],
    )(s)


# speedup vs baseline: 2.2608x; 1.0855x over previous
import numpy as np
import jax
import jax.numpy as jnp
from jax import lax
from jax.experimental import pallas as pl
from jax.experimental.pallas import tpu as pltpu, tpu_sc as plsc

NEURONS_C = 32768
K_C = 64
BATCH_C = 128
NWORKERS = 32
ROWS_PER_WORKER = BATCH_C // NWORKERS
NV_ROW = NEURONS_C // 16
NSEG = NV_ROW // 16

MIN32 = np.int32(-2**31)
M7F = np.int32(0x7FFFFFFF)
NEG_INF = np.float32(-np.inf)


def _keyf(u):
    sk = u ^ MIN32
    return lax.bitcast_convert_type(
        sk ^ (lax.shift_right_arithmetic(sk, 31) & M7F), jnp.float32)


def _sc_body(s_hbm, o_hbm, rowa_v, rowb_v, gmax_v, cmax_v, glist_v, cv_v,
             in0_sem, in1_sem, out0_sem, out1_sem):
    wid = lax.axis_index("s") * 2 + lax.axis_index("c")
    lane = lax.iota(jnp.int32, 16)
    lane16 = lane * 16
    zeros16 = jnp.zeros((16,), jnp.int32)
    ninf16 = jnp.full((16,), NEG_INF)

    def compute_row(buf, row):
        @plsc.parallel_loop(0, NSEG, unroll=2)
        def _g(t):
            vs = [buf[pl.ds(t * 256 + k * 16, 16)] for k in range(16)]
            while len(vs) > 1:
                vs = [jnp.maximum(vs[2 * a], vs[2 * a + 1])
                      for a in range(len(vs) // 2)]
            gmax_v[pl.ds(t * 16, 16)] = vs[0]

        for t in range(16):
            vs = [gmax_v[pl.ds(t * 128 + k * 16, 16)] for k in range(8)]
            while len(vs) > 1:
                vs = [jnp.maximum(vs[2 * a], vs[2 * a + 1])
                      for a in range(len(vs) // 2)]
            cmax_v[pl.ds(t * 16, 16)] = vs[0]

        u = zeros16
        for b in range(31, -1, -1):
            bit = MIN32 if b == 31 else np.int32(1 << b)
            t_f = _keyf(u | bit)

            @plsc.parallel_loop(0, 4, carry=zeros16)
            def cnt(t, acc_c):
                for q in range(4):
                    m = cmax_v[pl.ds(t * 64 + q * 16, 16)] >= t_f
                    acc_c = acc_c + plsc.all_reduce_population_count(m)
                return acc_c
            u = jnp.where(cnt >= K_C, u | bit, u)
        uc = u
        c64_f = _keyf(u)

        @plsc.parallel_loop(0, NSEG, unroll=4, carry=zeros16)
        def ng(t, c_c):
            m = gmax_v[pl.ds(t * 16, 16)] >= c64_f
            pos = c_c + plsc.cumsum(m.astype(jnp.int32)) - 1
            plsc.store_scatter(glist_v, [pos], lane + t * 16, mask=m)
            return c_c + plsc.all_reduce_population_count(m)
        ng_s = ng[0]

        @plsc.parallel_loop(0, ng_s, unroll=2)
        def _cg(g):
            gid = plsc.load_gather(glist_v, [g + zeros16])
            base = lax.shift_left(lax.shift_right_arithmetic(gid, 4), 8)
            idx = base + lane16 + (gid & 15)
            cv_v[pl.ds(g * 16, 16)] = plsc.load_gather(buf, [idx])

        for t in range(3):
            cv_v[pl.ds((ng_s + t) * 16, 16)] = ninf16
        nv4 = (ng_s + 3) >> 2

        def count_cmp(t_f, strict=False):
            @plsc.parallel_loop(0, nv4, unroll=2, carry=zeros16)
            def acc(t, acc_c):
                for q in range(4):
                    x = cv_v[pl.ds(t * 64 + q * 16, 16)]
                    m = (x > t_f) if strict else (x >= t_f)
                    acc_c = acc_c + plsc.all_reduce_population_count(m)
                return acc_c
            return acc

        mm = [cmax_v[pl.ds(k * 16, 16)] for k in range(16)]
        while len(mm) > 1:
            mm = [jnp.maximum(mm[2 * a], mm[2 * a + 1])
                  for a in range(len(mm) // 2)]
        hmax = jnp.max(mm[0]) + jnp.zeros((16,), jnp.float32)
        ih = lax.bitcast_convert_type(hmax, jnp.int32)
        uh = ih ^ (lax.shift_right_arithmetic(ih, 31) | MIN32)
        d = uc ^ uh
        df = d.astype(jnp.float32)
        hb = (lax.shift_right_logical(
            lax.bitcast_convert_type(df, jnp.int32), 23) & 255) - 127
        hb = jnp.where(d < 0, 31, hb)
        nb = jnp.maximum(hb + 1, 0)
        hbc = jnp.maximum(hb, 0)
        u0 = jnp.where(d == 0, uc, uc & ~(lax.shift_left(2, hbc) - 1))
        ones_i = jnp.ones((16,), jnp.int32)

        def _step(i, u):
            bit = lax.shift_left(ones_i, hb - i)
            cnt = count_cmp(_keyf(u | bit))
            return jnp.where(cnt >= K_C, u | bit, u)

        u = lax.fori_loop(0, nb[0], _step, u0)
        thr_f = _keyf(u)

        n_gt = count_cmp(thr_f, strict=True)
        need = K_C - n_gt
        n_eq = count_cmp(thr_f) - n_gt

        def tie_search():
            jcut0 = zeros16
            for b in range(14, -1, -1):
                candj = jcut0 | np.int32(1 << b)

                @plsc.parallel_loop(0, NV_ROW, unroll=4, carry=zeros16)
                def cj(i, c_c):
                    x = buf[pl.ds(i * 16, 16)]
                    m = (x == thr_f) & (lane + i * 16 >= candj)
                    return c_c + plsc.all_reduce_population_count(m)
                jcut0 = jnp.where(cj >= need, candj, jcut0)
            return jcut0

        tie_mode = ((n_eq != need).astype(jnp.int32))[0] != 0
        jcut = lax.cond(tie_mode, tie_search, lambda: zeros16)

        @pl.when(jnp.logical_not(tie_mode))
        def _mask_fast():
            @plsc.parallel_loop(0, NV_ROW, unroll=8)
            def _mk(i):
                x = buf[pl.ds(i * 16, 16)]
                buf[pl.ds(i * 16, 16)] = jnp.where(x >= thr_f, x, 0.0)

        @pl.when(tie_mode)
        def _mask_tie():
            @plsc.parallel_loop(0, NV_ROW, unroll=4)
            def _mk(i):
                x = buf[pl.ds(i * 16, 16)]
                keep = (x > thr_f) | ((x == thr_f) & (lane + i * 16 >= jcut))
                buf[pl.ds(i * 16, 16)] = jnp.where(keep, x, 0.0)

    base = wid * ROWS_PER_WORKER

    def in_dma(buf, row, sem):
        return pltpu.make_async_copy(s_hbm.at[row], buf, sem)

    def out_dma(buf, row, sem):
        return pltpu.make_async_copy(buf, o_hbm.at[row], sem)

    in_dma(rowa_v, base, in0_sem).start()

    def _pair(p, _):
        r0 = base + 2 * p
        in_dma(rowa_v, r0, in0_sem).wait()

        @pl.when(p > 0)
        def _w():
            out_dma(rowb_v, r0 - 1, out1_sem).wait()
        in_dma(rowb_v, r0 + 1, in1_sem).start()
        compute_row(rowa_v, r0)
        out_dma(rowa_v, r0, out0_sem).start()
        in_dma(rowb_v, r0 + 1, in1_sem).wait()
        compute_row(rowb_v, r0 + 1)
        out_dma(rowb_v, r0 + 1, out1_sem).start()
        out_dma(rowa_v, r0, out0_sem).wait()

        @pl.when(p == 0)
        def _n():
            in_dma(rowa_v, r0 + 2, in0_sem).start()
        return 0

    lax.fori_loop(0, ROWS_PER_WORKER // 2, _pair, 0)
    out_dma(rowb_v, base + 3, out1_sem).wait()


@jax.jit
def kernel(s):
    mesh = plsc.VectorSubcoreMesh(core_axis_name="c", subcore_axis_name="s",
                                  num_cores=2, num_subcores=16)
    return pl.kernel(
        _sc_body,
        out_type=jax.ShapeDtypeStruct((BATCH_C, NEURONS_C), jnp.float32),
        mesh=mesh,
        compiler_params=pltpu.CompilerParams(needs_layout_passes=False),
        scratch_types=[
            pltpu.VMEM((NEURONS_C,), jnp.float32),
            pltpu.VMEM((NEURONS_C,), jnp.float32),
            pltpu.VMEM((NV_ROW,), jnp.float32),
            pltpu.VMEM((256,), jnp.float32),
            pltpu.VMEM((NV_ROW + 16,), jnp.int32),
            pltpu.VMEM((NEURONS_C + 64,), jnp.float32),
            pltpu.SemaphoreType.DMA,
            pltpu.SemaphoreType.DMA,
            pltpu.SemaphoreType.DMA,
            pltpu.SemaphoreType.DMA,
        ],
    )(s)


# speedup vs baseline: 2.3651x; 1.0461x over previous
import numpy as np
import jax
import jax.numpy as jnp
from jax import lax
from jax.experimental import pallas as pl
from jax.experimental.pallas import tpu as pltpu, tpu_sc as plsc

NEURONS_C = 32768
K_C = 64
BATCH_C = 128
NWORKERS = 32
ROWS_PER_WORKER = BATCH_C // NWORKERS
NV_ROW = NEURONS_C // 16
NSEG = NV_ROW // 16

MIN32 = np.int32(-2**31)
M7F = np.int32(0x7FFFFFFF)
NEG_INF = np.float32(-np.inf)


def _keyf(u):
    sk = u ^ MIN32
    return lax.bitcast_convert_type(
        sk ^ (lax.shift_right_arithmetic(sk, 31) & M7F), jnp.float32)


def _sc_body(s_hbm, o_hbm, rowa_v, rowb_v, gmax_v, cmax_v, glist_v, cv_v,
             in0_sem, in1_sem, out0_sem, out1_sem):
    wid = lax.axis_index("s") * 2 + lax.axis_index("c")
    lane = lax.iota(jnp.int32, 16)
    lane16 = lane * 16
    zeros16 = jnp.zeros((16,), jnp.int32)
    ninf16 = jnp.full((16,), NEG_INF)

    def compute_row(buf, row, mid_cb=None):
        @plsc.parallel_loop(0, NSEG, unroll=2)
        def _g(t):
            vs = [buf[pl.ds(t * 256 + k * 16, 16)] for k in range(16)]
            while len(vs) > 1:
                vs = [jnp.maximum(vs[2 * a], vs[2 * a + 1])
                      for a in range(len(vs) // 2)]
            gmax_v[pl.ds(t * 16, 16)] = vs[0]

        for t in range(16):
            vs = [gmax_v[pl.ds(t * 128 + k * 16, 16)] for k in range(8)]
            while len(vs) > 1:
                vs = [jnp.maximum(vs[2 * a], vs[2 * a + 1])
                      for a in range(len(vs) // 2)]
            cmax_v[pl.ds(t * 16, 16)] = vs[0]

        if mid_cb is not None:
            mid_cb()

        u = zeros16
        for b in range(31, -1, -1):
            bit = MIN32 if b == 31 else np.int32(1 << b)
            t_f = _keyf(u | bit)
            cnt = zeros16
            for t in range(16):
                m = cmax_v[pl.ds(t * 16, 16)] >= t_f
                cnt = cnt + plsc.all_reduce_population_count(m)
            u = jnp.where(cnt >= K_C, u | bit, u)
        uc = u
        c64_f = _keyf(u)

        @plsc.parallel_loop(0, NSEG, unroll=4, carry=zeros16)
        def ng(t, c_c):
            m = gmax_v[pl.ds(t * 16, 16)] >= c64_f
            pos = c_c + plsc.cumsum(m.astype(jnp.int32)) - 1
            plsc.store_scatter(glist_v, [pos], lane + t * 16, mask=m)
            return c_c + plsc.all_reduce_population_count(m)
        ng_s = ng[0]

        @plsc.parallel_loop(0, ng_s, unroll=2)
        def _cg(g):
            gid = plsc.load_gather(glist_v, [g + zeros16])
            base = lax.shift_left(lax.shift_right_arithmetic(gid, 4), 8)
            idx = base + lane16 + (gid & 15)
            cv_v[pl.ds(g * 16, 16)] = plsc.load_gather(buf, [idx])

        for t in range(3):
            cv_v[pl.ds((ng_s + t) * 16, 16)] = ninf16
        nv4 = (ng_s + 3) >> 2

        def count_cmp(t_f, strict=False):
            @plsc.parallel_loop(0, nv4, unroll=2, carry=zeros16)
            def acc(t, acc_c):
                for q in range(4):
                    x = cv_v[pl.ds(t * 64 + q * 16, 16)]
                    m = (x > t_f) if strict else (x >= t_f)
                    acc_c = acc_c + plsc.all_reduce_population_count(m)
                return acc_c
            return acc

        mm = [cmax_v[pl.ds(k * 16, 16)] for k in range(16)]
        while len(mm) > 1:
            mm = [jnp.maximum(mm[2 * a], mm[2 * a + 1])
                  for a in range(len(mm) // 2)]
        hmax = jnp.max(mm[0]) + jnp.zeros((16,), jnp.float32)
        ih = lax.bitcast_convert_type(hmax, jnp.int32)
        uh = ih ^ (lax.shift_right_arithmetic(ih, 31) | MIN32)
        d = uc ^ uh
        df = d.astype(jnp.float32)
        hb = (lax.shift_right_logical(
            lax.bitcast_convert_type(df, jnp.int32), 23) & 255) - 127
        hb = jnp.where(d < 0, 31, hb)
        nb = jnp.maximum(hb + 1, 0)
        hbc = jnp.maximum(hb, 0)
        u0 = jnp.where(d == 0, uc, uc & ~(lax.shift_left(2, hbc) - 1))
        ones_i = jnp.ones((16,), jnp.int32)

        def _step(i, u):
            bit = lax.shift_left(ones_i, hb - i)
            cnt = count_cmp(_keyf(u | bit))
            return jnp.where(cnt >= K_C, u | bit, u)

        u = lax.fori_loop(0, nb[0], _step, u0)
        thr_f = _keyf(u)

        n_gt = count_cmp(thr_f, strict=True)
        need = K_C - n_gt
        n_eq = count_cmp(thr_f) - n_gt

        def tie_search():
            jcut0 = zeros16
            for b in range(14, -1, -1):
                candj = jcut0 | np.int32(1 << b)

                @plsc.parallel_loop(0, NV_ROW, unroll=4, carry=zeros16)
                def cj(i, c_c):
                    x = buf[pl.ds(i * 16, 16)]
                    m = (x == thr_f) & (lane + i * 16 >= candj)
                    return c_c + plsc.all_reduce_population_count(m)
                jcut0 = jnp.where(cj >= need, candj, jcut0)
            return jcut0

        tie_mode = ((n_eq != need).astype(jnp.int32))[0] != 0
        jcut = lax.cond(tie_mode, tie_search, lambda: zeros16)

        @pl.when(jnp.logical_not(tie_mode))
        def _mask_fast():
            @plsc.parallel_loop(0, NV_ROW, unroll=8)
            def _mk(i):
                x = buf[pl.ds(i * 16, 16)]
                buf[pl.ds(i * 16, 16)] = jnp.where(x >= thr_f, x, 0.0)

        @pl.when(tie_mode)
        def _mask_tie():
            @plsc.parallel_loop(0, NV_ROW, unroll=4)
            def _mk(i):
                x = buf[pl.ds(i * 16, 16)]
                keep = (x > thr_f) | ((x == thr_f) & (lane + i * 16 >= jcut))
                buf[pl.ds(i * 16, 16)] = jnp.where(keep, x, 0.0)

    base = wid * ROWS_PER_WORKER

    def in_dma(buf, row, sem):
        return pltpu.make_async_copy(s_hbm.at[row], buf, sem)

    def out_dma(buf, row, sem):
        return pltpu.make_async_copy(buf, o_hbm.at[row], sem)

    in_dma(rowa_v, base, in0_sem).start()

    def _pair(p, _):
        r0 = base + 2 * p
        in_dma(rowa_v, r0, in0_sem).wait()

        @pl.when(p > 0)
        def _w():
            out_dma(rowb_v, r0 - 1, out1_sem).wait()
        in_dma(rowb_v, r0 + 1, in1_sem).start()
        compute_row(rowa_v, r0)
        out_dma(rowa_v, r0, out0_sem).start()
        in_dma(rowb_v, r0 + 1, in1_sem).wait()

        def _recycle_a():
            out_dma(rowa_v, r0, out0_sem).wait()

            @pl.when(p == 0)
            def _n():
                in_dma(rowa_v, r0 + 2, in0_sem).start()

        compute_row(rowb_v, r0 + 1, mid_cb=_recycle_a)
        out_dma(rowb_v, r0 + 1, out1_sem).start()
        return 0

    lax.fori_loop(0, ROWS_PER_WORKER // 2, _pair, 0)
    out_dma(rowb_v, base + 3, out1_sem).wait()


@jax.jit
def kernel(s):
    mesh = plsc.VectorSubcoreMesh(core_axis_name="c", subcore_axis_name="s",
                                  num_cores=2, num_subcores=16)
    return pl.kernel(
        _sc_body,
        out_type=jax.ShapeDtypeStruct((BATCH_C, NEURONS_C), jnp.float32),
        mesh=mesh,
        compiler_params=pltpu.CompilerParams(needs_layout_passes=False),
        scratch_types=[
            pltpu.VMEM((NEURONS_C,), jnp.float32),
            pltpu.VMEM((NEURONS_C,), jnp.float32),
            pltpu.VMEM((NV_ROW,), jnp.float32),
            pltpu.VMEM((256,), jnp.float32),
            pltpu.VMEM((NV_ROW + 16,), jnp.int32),
            pltpu.VMEM((NEURONS_C + 64,), jnp.float32),
            pltpu.SemaphoreType.DMA,
            pltpu.SemaphoreType.DMA,
            pltpu.SemaphoreType.DMA,
            pltpu.SemaphoreType.DMA,
        ],
    )(s)
